# Initial kernel scaffold; baseline (speedup 1.0000x reference)
import functools

import jax
import jax.numpy as jnp
from jax import lax
from jax.experimental import pallas as pl
from jax.experimental.pallas import tpu as pltpu
from jax.experimental.pallas import tpu_sc as plsc

DIM = 128
SLOPE = 0.1
NW = 32
LANES = 16
EB = 128


def _dense_body(f_ref, w_ref, b_ref, at_ref, ab_ref, ne_ref, sc_ref):
    ne = jnp.dot(f_ref[...], w_ref[...], preferred_element_type=jnp.float32)
    ne = ne + b_ref[...]
    ne_ref[...] = ne
    sc_ref[0, :] = jnp.sum(ne * at_ref[...], axis=1)
    sc_ref[1, :] = jnp.sum(ne * ab_ref[...], axis=1)


def _combine_body(a0_ref, a1_ref, rs_ref, o_ref):
    tot = a0_ref[...] + a1_ref[...]
    r = jnp.sum(rs_ref[...], axis=0)
    o_ref[...] = tot / r[:, None]


def _make_sc_kernel(npad, steps):
    rows_per_tile = npad // LANES
    mesh = plsc.VectorSubcoreMesh(core_axis_name="c", subcore_axis_name="s")

    @functools.partial(
        pl.kernel,
        out_type=(
            jax.ShapeDtypeStruct((2, npad, DIM), jnp.float32),
            jax.ShapeDtypeStruct((NW, npad), jnp.float32),
        ),
        mesh=mesh,
        scratch_types=[
            pltpu.VMEM((npad,), jnp.float32),
            pltpu.VMEM((npad,), jnp.float32),
            pltpu.VMEM((npad,), jnp.float32),
            pltpu.VMEM((steps, EB), jnp.int32),
            pltpu.VMEM((steps, EB), jnp.int32),
            pltpu.VMEM((EB, DIM), jnp.float32),
            pltpu.VMEM((EB,), jnp.float32),
            pltpu.VMEM_SHARED((npad, DIM), jnp.float32),
            pltpu.SemaphoreType.DMA,
        ],
    )
    def sc_kernel(ne_hbm, s_hbm, d_hbm, src_hbm, dst_hbm,
                  acc_out, rs_out,
                  s_tab, d_tab, rs_local, src_c, dst_c, rows, vals, acc, sem):
        cid = lax.axis_index("c")
        sid = lax.axis_index("s")
        wid = sid * 2 + cid

        pltpu.sync_copy(s_hbm, s_tab)
        pltpu.sync_copy(d_hbm, d_tab)
        pltpu.sync_copy(src_hbm.at[wid], src_c)
        pltpu.sync_copy(dst_hbm.at[wid], dst_c)

        zero16 = jnp.zeros((LANES,), jnp.float32)

        def zrow(j, carry):
            for c8 in range(DIM // LANES):
                rows[j, pl.ds(c8 * LANES, LANES)] = zero16
            return carry
        lax.fori_loop(0, EB, zrow, 0)

        def zrs(j, carry):
            rs_local[pl.ds(j * LANES, LANES)] = zero16
            return carry
        lax.fori_loop(0, npad // LANES, zrs, 0)

        for t in range(rows_per_tile // EB):
            pltpu.sync_copy(rows, acc.at[pl.ds(sid * rows_per_tile + t * EB, EB)])
        plsc.subcore_barrier()

        def step_fn(st, carry):
            pltpu.async_copy(ne_hbm.at[dst_c.at[st]], rows, sem).wait()
            for g in range(EB // LANES):
                isrc = src_c[st, pl.ds(g * LANES, LANES)]
                idst = dst_c[st, pl.ds(g * LANES, LANES)]
                sv = plsc.load_gather(s_tab, [isrc])
                dv = plsc.load_gather(d_tab, [idst])
                x = sv + dv
                v = jnp.exp(jnp.maximum(x, x * SLOPE))
                vals[pl.ds(g * LANES, LANES)] = v
                plsc.addupdate_scatter(rs_local, [isrc], v)

            def scale(j, carry2):
                vv = jnp.full((LANES,), vals[j], dtype=jnp.float32)
                for c8 in range(DIM // LANES):
                    sl = pl.ds(c8 * LANES, LANES)
                    rows[j, sl] = rows[j, sl] * vv
                return carry2
            lax.fori_loop(0, EB, scale, 0)

            pltpu.sync_copy(rows, acc.at[src_c.at[st]], add=True)
            return carry
        lax.fori_loop(0, steps, step_fn, 0)

        plsc.subcore_barrier()
        pltpu.sync_copy(acc.at[pl.ds(sid * rows_per_tile, rows_per_tile)],
                        acc_out.at[cid, pl.ds(sid * rows_per_tile, rows_per_tile)])
        pltpu.sync_copy(rs_local, rs_out.at[wid])

    return sc_kernel


def kernel(features, W, b, a, nodes, edge_index, ind):
    n = features.shape[0]
    npad = -((n + 1) // -(LANES * EB)) * (LANES * EB)

    n_edges = edge_index.shape[1] + nodes.shape[0]
    steps = -(n_edges // -(NW * EB))
    epad = NW * steps * EB

    feat_pad = jnp.pad(features, ((0, npad - n), (0, 0)))
    a_top = a[:DIM, 0].reshape(1, DIM)
    a_bot = a[DIM:, 0].reshape(1, DIM)
    blk = 1024
    ne, scores = pl.pallas_call(
        _dense_body,
        grid=(npad // blk,),
        in_specs=[
            pl.BlockSpec((blk, DIM), lambda i: (i, 0)),
            pl.BlockSpec((DIM, DIM), lambda i: (0, 0)),
            pl.BlockSpec((1, DIM), lambda i: (0, 0)),
            pl.BlockSpec((1, DIM), lambda i: (0, 0)),
            pl.BlockSpec((1, DIM), lambda i: (0, 0)),
        ],
        out_specs=[
            pl.BlockSpec((blk, DIM), lambda i: (i, 0)),
            pl.BlockSpec((2, blk), lambda i: (0, i)),
        ],
        out_shape=[
            jax.ShapeDtypeStruct((npad, DIM), jnp.float32),
            jax.ShapeDtypeStruct((2, npad), jnp.float32),
        ],
    )(feat_pad, W, b.reshape(1, DIM), a_top, a_bot)

    pad_e = epad - n_edges
    src = jnp.concatenate(
        [edge_index[0], nodes, jnp.full((pad_e,), n, jnp.int32)]).astype(jnp.int32)
    dst = jnp.concatenate(
        [edge_index[1], nodes, jnp.zeros((pad_e,), jnp.int32)]).astype(jnp.int32)
    srcm = src.reshape(NW, steps, EB)
    dstm = dst.reshape(NW, steps, EB)

    acc, rs = _make_sc_kernel(npad, steps)(ne, scores[0], scores[1], srcm, dstm)

    out_pad = pl.pallas_call(
        _combine_body,
        grid=(npad // blk,),
        in_specs=[
            pl.BlockSpec((blk, DIM), lambda i: (i, 0)),
            pl.BlockSpec((blk, DIM), lambda i: (i, 0)),
            pl.BlockSpec((NW, blk), lambda i: (0, i)),
        ],
        out_specs=pl.BlockSpec((blk, DIM), lambda i: (i, 0)),
        out_shape=jax.ShapeDtypeStruct((npad, DIM), jnp.float32),
    )(acc[0], acc[1], rs)
    return out_pad[:n]


# speedup vs baseline: 9.1560x; 9.1560x over previous
import functools

import jax
import jax.numpy as jnp
from jax import lax
from jax.experimental import pallas as pl
from jax.experimental.pallas import tpu as pltpu
from jax.experimental.pallas import tpu_sc as plsc

DIM = 128
SLOPE = 0.1
NW = 32
LANES = 16
EB = 128


def _dense_body(f_ref, w_ref, b_ref, at_ref, ab_ref, ne_ref, sc_ref):
    ne = jnp.dot(f_ref[...], w_ref[...], preferred_element_type=jnp.float32)
    ne = ne + b_ref[...]
    ne_ref[...] = ne
    sc_ref[0, :] = jnp.sum(ne * at_ref[...], axis=1)
    sc_ref[1, :] = jnp.sum(ne * ab_ref[...], axis=1)


def _combine_body(a0_ref, a1_ref, rs_ref, o_ref):
    tot = a0_ref[...] + a1_ref[...]
    r = jnp.sum(rs_ref[...], axis=0)
    o_ref[...] = tot / r[:, None]


def _make_sc_kernel(npad, steps):
    rows_per_tile = npad // LANES
    mesh = plsc.VectorSubcoreMesh(core_axis_name="c", subcore_axis_name="s")

    @functools.partial(
        pl.kernel,
        out_type=(
            jax.ShapeDtypeStruct((2, npad, DIM), jnp.float32),
            jax.ShapeDtypeStruct((2, npad), jnp.float32),
        ),
        mesh=mesh,
        scratch_types=[
            pltpu.VMEM((npad,), jnp.float32),
            pltpu.VMEM((npad,), jnp.float32),
            pltpu.VMEM((EB,), jnp.int32),
            pltpu.VMEM((EB,), jnp.int32),
            pltpu.VMEM((EB, DIM), jnp.float32),
            pltpu.VMEM((EB,), jnp.float32),
            pltpu.VMEM((npad // LANES,), jnp.float32),
            pltpu.VMEM_SHARED((npad, DIM), jnp.float32),
            pltpu.VMEM_SHARED((npad,), jnp.float32),
            pltpu.SemaphoreType.DMA,
        ],
        compiler_params=pltpu.CompilerParams(needs_layout_passes=False),
    )
    def sc_kernel(ne_hbm, s_hbm, d_hbm, src_hbm, dst_hbm,
                  acc_out, rs_out,
                  s_tab, d_tab, src_b, dst_b, rows, vals, zeros1d,
                  acc, rs_sh, sem):
        cid = lax.axis_index("c")
        sid = lax.axis_index("s")
        wid = sid * 2 + cid

        pltpu.sync_copy(s_hbm, s_tab)
        pltpu.sync_copy(d_hbm, d_tab)

        zero16 = jnp.zeros((LANES,), jnp.float32)

        def zrow(j, carry):
            for c8 in range(DIM // LANES):
                rows[j, pl.ds(c8 * LANES, LANES)] = zero16
            return carry
        lax.fori_loop(0, EB, zrow, 0)

        def zz(j, carry):
            zeros1d[pl.ds(j * LANES, LANES)] = zero16
            return carry
        lax.fori_loop(0, rows_per_tile // LANES, zz, 0)

        for t in range(rows_per_tile // EB):
            pltpu.sync_copy(rows, acc.at[pl.ds(sid * rows_per_tile + t * EB, EB)])
        pltpu.sync_copy(zeros1d, rs_sh.at[pl.ds(sid * rows_per_tile, rows_per_tile)])
        plsc.subcore_barrier()

        def step_fn(st, carry):
            pltpu.sync_copy(src_hbm.at[wid, st], src_b)
            pltpu.sync_copy(dst_hbm.at[wid, st], dst_b)
            pltpu.async_copy(ne_hbm.at[dst_b], rows, sem).wait()
            for g in range(EB // LANES):
                isrc = src_b[pl.ds(g * LANES, LANES)]
                idst = dst_b[pl.ds(g * LANES, LANES)]
                sv = plsc.load_gather(s_tab, [isrc])
                dv = plsc.load_gather(d_tab, [idst])
                x = sv + dv
                v = jnp.exp(jnp.maximum(x, x * SLOPE))
                vals[pl.ds(g * LANES, LANES)] = v

            def scale(gg, carry2):
                vg = vals[pl.ds(gg * LANES, LANES)]
                for l in range(LANES):
                    vv = jnp.broadcast_to(vg[l], (LANES,))
                    j = gg * LANES + l
                    for c8 in range(DIM // LANES):
                        sl = pl.ds(c8 * LANES, LANES)
                        rows[j, sl] = rows[j, sl] * vv
                return carry2
            lax.fori_loop(0, EB // LANES, scale, 0)

            pltpu.sync_copy(rows, acc.at[src_b], add=True)
            pltpu.sync_copy(vals, rs_sh.at[src_b], add=True)
            return carry
        lax.fori_loop(0, steps, step_fn, 0)

        plsc.subcore_barrier()
        pltpu.sync_copy(acc.at[pl.ds(sid * rows_per_tile, rows_per_tile)],
                        acc_out.at[cid, pl.ds(sid * rows_per_tile, rows_per_tile)])
        pltpu.sync_copy(rs_sh.at[pl.ds(sid * rows_per_tile, rows_per_tile)],
                        rs_out.at[cid, pl.ds(sid * rows_per_tile, rows_per_tile)])

    return sc_kernel


def kernel(features, W, b, a, nodes, edge_index, ind):
    n = features.shape[0]
    npad = -((n + 1) // -(LANES * EB)) * (LANES * EB)

    n_edges = edge_index.shape[1] + nodes.shape[0]
    steps = -(n_edges // -(NW * EB))
    epad = NW * steps * EB

    feat_pad = jnp.pad(features, ((0, npad - n), (0, 0)))
    a_top = a[:DIM, 0].reshape(1, DIM)
    a_bot = a[DIM:, 0].reshape(1, DIM)
    blk = 1024
    ne, scores = pl.pallas_call(
        _dense_body,
        grid=(npad // blk,),
        in_specs=[
            pl.BlockSpec((blk, DIM), lambda i: (i, 0)),
            pl.BlockSpec((DIM, DIM), lambda i: (0, 0)),
            pl.BlockSpec((1, DIM), lambda i: (0, 0)),
            pl.BlockSpec((1, DIM), lambda i: (0, 0)),
            pl.BlockSpec((1, DIM), lambda i: (0, 0)),
        ],
        out_specs=[
            pl.BlockSpec((blk, DIM), lambda i: (i, 0)),
            pl.BlockSpec((2, blk), lambda i: (0, i)),
        ],
        out_shape=[
            jax.ShapeDtypeStruct((npad, DIM), jnp.float32),
            jax.ShapeDtypeStruct((2, npad), jnp.float32),
        ],
    )(feat_pad, W, b.reshape(1, DIM), a_top, a_bot)

    pad_e = epad - n_edges
    src = jnp.concatenate(
        [edge_index[0], nodes, jnp.full((pad_e,), n, jnp.int32)]).astype(jnp.int32)
    dst = jnp.concatenate(
        [edge_index[1], nodes, jnp.zeros((pad_e,), jnp.int32)]).astype(jnp.int32)
    srcm = src.reshape(NW, steps, EB)
    dstm = dst.reshape(NW, steps, EB)

    acc, rs = _make_sc_kernel(npad, steps)(ne, scores[0], scores[1], srcm, dstm)

    out_pad = pl.pallas_call(
        _combine_body,
        grid=(npad // blk,),
        in_specs=[
            pl.BlockSpec((blk, DIM), lambda i: (i, 0)),
            pl.BlockSpec((blk, DIM), lambda i: (i, 0)),
            pl.BlockSpec((2, blk), lambda i: (0, i)),
        ],
        out_specs=pl.BlockSpec((blk, DIM), lambda i: (i, 0)),
        out_shape=jax.ShapeDtypeStruct((npad, DIM), jnp.float32),
    )(acc[0], acc[1], rs)
    return out_pad[:n]
